# baseline (speedup 1.0000x reference)
import jax
import jax.numpy as jnp
from jax.experimental import pallas as pl
from jax.experimental.pallas import tpu as pltpu

_B = 1024
_S = 64
_D = 1024
_H = 2048
_NE = 8
_R = 128
_T = _B // _R
_G = 2 * _NE
_EPS = 1e-5
_NC = 16

_f32 = jnp.float32


def _routing(key16):
    onehot16 = (key16[:, None] == jnp.arange(_G, dtype=jnp.int32)[None, :]
                ).astype(jnp.int32)
    cum = jnp.cumsum(onehot16, axis=0)
    counts16 = cum[-1]
    starts16 = jnp.cumsum(counts16) - counts16
    rank = jnp.sum(onehot16 * (cum - 1), axis=1)
    base = jnp.sum(onehot16 * starts16[None, :], axis=1)
    oinv = (base + rank).astype(jnp.int32)

    def tables(counts):
        starts = jnp.cumsum(counts) - counts
        ends = starts + counts
        tlo = starts // _R
        thi = (ends + _R - 1) // _R
        ntl = jnp.where(counts > 0, thi - tlo, 0)
        return (tlo.astype(jnp.int32), ntl.astype(jnp.int32),
                starts.astype(jnp.int32), ends.astype(jnp.int32))

    counts8 = counts16[0::2] + counts16[1::2]
    return oinv, tables(counts16), tables(counts8)


def _gelu(x):
    return 0.5 * x * (1.0 + jax.lax.erf(x * 0.7071067811865476))


def _bf(x):
    return x.astype(jnp.bfloat16)


def _copy_expert(hbm_ref, slots_ref, sems_ref, k, start):
    slot = jax.lax.rem(k, 2)
    rows = hbm_ref.shape[1]
    c_rows = rows // _NC
    for c in range(_NC):
        cp = pltpu.make_async_copy(
            hbm_ref.at[k, pl.ds(c * c_rows, c_rows), :],
            slots_ref.at[slot, pl.ds(c * c_rows, c_rows), :],
            sems_ref.at[slot, c])
        if start:
            cp.start()
        else:
            cp.wait()


def _stage_a_body(s_tlo, s_ntl, s_lo, s_hi,
                  oinv_ref, x_ref,
                  bph1_ref, bph2_ref, bpr1_ref, bpr2_ref,
                  lng_ref, lnb_ref,
                  wph1_hbm, wpr1_hbm, wph2_hbm, wpr2_hbm,
                  out_ref,
                  xs_all, w1ph, w1pr, ph_slots, pr_slots,
                  w1_sems, ph_sems, pr_sems):
    k = pl.program_id(0)

    def copy_w1(hbm_ref, dst, sem_col, start):
        for c in range(2):
            cp = pltpu.make_async_copy(
                hbm_ref.at[pl.ds(c * (_NE // 2), _NE // 2)],
                dst.at[pl.ds(c * (_NE // 2), _NE // 2)],
                w1_sems.at[sem_col, c])
            if start:
                cp.start()
            else:
                cp.wait()

    @pl.when(k == 0)
    def _():
        copy_w1(wph1_hbm, w1ph, 0, True)
        copy_w1(wpr1_hbm, w1pr, 1, True)
        _copy_expert(wph2_hbm, ph_slots, ph_sems, 0, True)
        _copy_expert(wpr2_hbm, pr_slots, pr_sems, 0, True)
        gat = (oinv_ref[...] ==
               jax.lax.broadcasted_iota(jnp.int32, (_B, 1), 0)).astype(_f32)
        xs_all[...] = jnp.dot(gat, x_ref[...], preferred_element_type=_f32)
        copy_w1(wph1_hbm, w1ph, 0, False)
        copy_w1(wpr1_hbm, w1pr, 1, False)

    @pl.when(k + 1 < _NE)
    def _():
        _copy_expert(wph2_hbm, ph_slots, ph_sems, k + 1, True)
        _copy_expert(wpr2_hbm, pr_slots, pr_sems, k + 1, True)

    def run(g, w1_full, b1_ref, w2_hbm, slots, sems, b2_ref):
        _copy_expert(w2_hbm, slots, sems, k, False)
        slot = jax.lax.rem(k, 2)
        w1b = _bf(w1_full[k])
        w2b = _bf(slots[slot])
        b1 = b1_ref[k]
        b2 = b2_ref[k]
        lng = lng_ref[k]
        lnb = lnb_ref[k]
        lo = s_lo[g]
        hi = s_hi[g]
        tlo = s_tlo[g]

        def tile(it, _):
            t = tlo + it
            p0 = t * _R + jax.lax.broadcasted_iota(jnp.int32, (_R, 1), 0)
            xs = xs_all[pl.ds(t * _R, _R), :]
            h = _gelu(jnp.dot(_bf(xs), w1b, preferred_element_type=_f32) + b1)
            y = jnp.dot(_bf(h), w2b, preferred_element_type=_f32) + b2
            mu = jnp.mean(y, axis=1, keepdims=True)
            var = jnp.mean(jnp.square(y - mu), axis=1, keepdims=True)
            yn = (y - mu) * jax.lax.rsqrt(var + _EPS) * lng + lnb
            gmask = (p0 >= lo) & (p0 < hi)
            out_ref[pl.ds(t * _R, _R), :] = jnp.where(
                gmask, yn, out_ref[pl.ds(t * _R, _R), :])
            return 0

        jax.lax.fori_loop(0, s_ntl[g], tile, 0)

    run(2 * k, w1ph, bph1_ref, wph2_hbm, ph_slots, ph_sems, bph2_ref)
    run(2 * k + 1, w1pr, bpr1_ref, wpr2_hbm, pr_slots, pr_sems, bpr2_ref)


def _stage_b_body(s_tlo, s_ntl, s_lo, s_hi,
                  oinvc_ref, xin_ref, bt1_ref, bt2_ref,
                  wt1_hbm, wt2_hbm,
                  out_ref,
                  ysort, t1_slots, t2_slots, t1_sems, t2_sems):
    e = pl.program_id(0)

    @pl.when(e == 0)
    def _():
        _copy_expert(wt1_hbm, t1_slots, t1_sems, 0, True)
        _copy_expert(wt2_hbm, t2_slots, t2_sems, 0, True)

    @pl.when(e + 1 < _NE)
    def _():
        _copy_expert(wt1_hbm, t1_slots, t1_sems, e + 1, True)
        _copy_expert(wt2_hbm, t2_slots, t2_sems, e + 1, True)

    _copy_expert(wt1_hbm, t1_slots, t1_sems, e, False)
    _copy_expert(wt2_hbm, t2_slots, t2_sems, e, False)
    slot = jax.lax.rem(e, 2)
    w1b = _bf(t1_slots[slot])
    w2b = _bf(t2_slots[slot])
    b1 = bt1_ref[e]
    b2 = bt2_ref[e]
    lo = s_lo[e]
    hi = s_hi[e]

    def tile(it, _):
        t = s_tlo[e] + it
        xs = xin_ref[pl.ds(t * _R, _R), :]
        h = _gelu(jnp.dot(_bf(xs), w1b, preferred_element_type=_f32) + b1)
        y = jnp.dot(_bf(h), w2b, preferred_element_type=_f32) + b2
        p0 = t * _R + jax.lax.broadcasted_iota(jnp.int32, (_R, 1), 0)
        gmask = (p0 >= lo) & (p0 < hi)
        ysort[pl.ds(t * _R, _R), :] = jnp.where(
            gmask, y, ysort[pl.ds(t * _R, _R), :])
        return 0

    jax.lax.fori_loop(0, s_ntl[e], tile, 0)

    @pl.when(e == _NE - 1)
    def _():
        scat = (oinvc_ref[...] ==
                jax.lax.broadcasted_iota(jnp.int32, (1, _B), 1)).astype(_f32)
        out_ref[...] = jnp.dot(scat, ysort[...], preferred_element_type=_f32)


_VMEM_FULL = lambda: pl.BlockSpec(memory_space=pltpu.MemorySpace.VMEM)
_HBM = lambda: pl.BlockSpec(memory_space=pltpu.MemorySpace.HBM)


@jax.jit
def kernel(raw_state, has_proprio, embodiment_idx, W_ph1, b_ph1, W_ph2, b_ph2,
           W_pr1, b_pr1, W_pr2, b_pr2, ln_g, ln_b, W_t1, b_t1, W_t2, b_t2):
    key16 = (embodiment_idx.astype(jnp.int32) * 2
             + has_proprio.astype(jnp.int32))
    oinv, (tlo16, ntl16, lo16, hi16), (tlo8, ntl8, lo8, hi8) = _routing(key16)

    mixed_ln = pl.pallas_call(
        _stage_a_body,
        grid_spec=pltpu.PrefetchScalarGridSpec(
            num_scalar_prefetch=4,
            grid=(_NE,),
            in_specs=[_VMEM_FULL() for _ in range(8)]
                     + [_HBM(), _HBM(), _HBM(), _HBM()],
            out_specs=_VMEM_FULL(),
            scratch_shapes=[
                pltpu.VMEM((_B, _S), _f32),
                pltpu.VMEM((_NE, _S, _H), _f32),
                pltpu.VMEM((_NE, _S, _H), _f32),
                pltpu.VMEM((2, _H, _D), _f32),
                pltpu.VMEM((2, _H, _D), _f32),
                pltpu.SemaphoreType.DMA((2, 2)),
                pltpu.SemaphoreType.DMA((2, _NC)),
                pltpu.SemaphoreType.DMA((2, _NC)),
            ],
        ),
        out_shape=jax.ShapeDtypeStruct((_B, _D), _f32),
        compiler_params=pltpu.CompilerParams(
            dimension_semantics=("arbitrary",),
            vmem_limit_bytes=120 * 1024 * 1024,
        ),
    )(tlo16, ntl16, lo16, hi16,
      oinv.reshape(1, _B), raw_state,
      b_ph1[:, None, :], b_ph2[:, None, :],
      b_pr1[:, None, :], b_pr2[:, None, :],
      ln_g[:, None, :], ln_b[:, None, :],
      W_ph1, W_pr1, W_ph2, W_pr2)

    out = pl.pallas_call(
        _stage_b_body,
        grid_spec=pltpu.PrefetchScalarGridSpec(
            num_scalar_prefetch=4,
            grid=(_NE,),
            in_specs=[_VMEM_FULL() for _ in range(4)] + [_HBM(), _HBM()],
            out_specs=_VMEM_FULL(),
            scratch_shapes=[
                pltpu.VMEM((_B, _D), _f32),
                pltpu.VMEM((2, _D, _H), _f32),
                pltpu.VMEM((2, _H, _D), _f32),
                pltpu.SemaphoreType.DMA((2, _NC)),
                pltpu.SemaphoreType.DMA((2, _NC)),
            ],
        ),
        out_shape=jax.ShapeDtypeStruct((_B, _D), _f32),
        compiler_params=pltpu.CompilerParams(
            dimension_semantics=("arbitrary",),
            vmem_limit_bytes=120 * 1024 * 1024,
        ),
    )(tlo8, ntl8, lo8, hi8,
      oinv.reshape(_B, 1), mixed_ln,
      b_t1[:, None, :], b_t2[:, None, :],
      W_t1, W_t2)

    return out[:, None, :]


# speedup vs baseline: 1.0234x; 1.0234x over previous
import jax
import jax.numpy as jnp
from jax.experimental import pallas as pl
from jax.experimental.pallas import tpu as pltpu

_B = 1024
_S = 64
_D = 1024
_H = 2048
_NE = 8
_R = 128
_T = _B // _R
_G = 2 * _NE
_EPS = 1e-5
_NC = 16

_f32 = jnp.float32


def _routing(key16):
    onehot16 = (key16[:, None] == jnp.arange(_G, dtype=jnp.int32)[None, :]
                ).astype(jnp.int32)
    cum = jnp.cumsum(onehot16, axis=0)
    counts16 = cum[-1]
    starts16 = jnp.cumsum(counts16) - counts16
    rank = jnp.sum(onehot16 * (cum - 1), axis=1)
    base = jnp.sum(onehot16 * starts16[None, :], axis=1)
    oinv = (base + rank).astype(jnp.int32)

    def tables(counts):
        starts = jnp.cumsum(counts) - counts
        ends = starts + counts
        tlo = starts // _R
        thi = (ends + _R - 1) // _R
        ntl = jnp.where(counts > 0, thi - tlo, 0)
        return (tlo.astype(jnp.int32), ntl.astype(jnp.int32),
                starts.astype(jnp.int32), ends.astype(jnp.int32))

    counts8 = counts16[0::2] + counts16[1::2]
    return oinv, tables(counts16), tables(counts8)


def _gelu(x):
    return 0.5 * x * (1.0 + jax.lax.erf(x * 0.7071067811865476))


def _bf(x):
    return x.astype(jnp.bfloat16)


def _copy_expert(hbm_ref, slots_ref, sems_ref, k, start):
    slot = jax.lax.rem(k, 2)
    rows = hbm_ref.shape[1]
    c_rows = rows // _NC
    for c in range(_NC):
        cp = pltpu.make_async_copy(
            hbm_ref.at[k, pl.ds(c * c_rows, c_rows), :],
            slots_ref.at[slot, pl.ds(c * c_rows, c_rows), :],
            sems_ref.at[slot, c])
        if start:
            cp.start()
        else:
            cp.wait()


def _stage_a_body(s_tlo, s_ntl, s_lo, s_hi,
                  oinv_ref, x_ref, wph1_ref, wpr1_ref,
                  bph1_ref, bph2_ref, bpr1_ref, bpr2_ref,
                  lng_ref, lnb_ref,
                  wph2_hbm, wpr2_hbm,
                  out_ref,
                  xs_all, ph_slots, pr_slots, ph_sems, pr_sems):
    g = pl.program_id(0)
    f = jax.lax.rem(g, 2)
    k = g // 2

    @pl.when(g == 0)
    def _():
        _copy_expert(wph2_hbm, ph_slots, ph_sems, 0, True)
        _copy_expert(wpr2_hbm, pr_slots, pr_sems, 0, True)
        gat = (oinv_ref[...] ==
               jax.lax.broadcasted_iota(jnp.int32, (_B, 1), 0)).astype(_f32)
        xs_all[...] = jnp.dot(gat, x_ref[...], preferred_element_type=_f32)

    @pl.when((f == 0) & (k + 1 < _NE))
    def _():
        _copy_expert(wph2_hbm, ph_slots, ph_sems, k + 1, True)

    @pl.when((f == 1) & (k + 1 < _NE))
    def _():
        _copy_expert(wpr2_hbm, pr_slots, pr_sems, k + 1, True)

    def run(w1_ref, b1_ref, w2_hbm, slots, sems, b2_ref):
        _copy_expert(w2_hbm, slots, sems, k, False)
        slot = jax.lax.rem(k, 2)
        w1b = _bf(w1_ref[k])
        w2b = _bf(slots[slot])
        b1 = b1_ref[k]
        b2 = b2_ref[k]
        lng = lng_ref[k]
        lnb = lnb_ref[k]
        lo = s_lo[g]
        hi = s_hi[g]

        def tile(it, _):
            t = s_tlo[g] + it
            p0 = t * _R + jax.lax.broadcasted_iota(jnp.int32, (_R, 1), 0)
            xs = xs_all[pl.ds(t * _R, _R), :]
            h = _gelu(jnp.dot(_bf(xs), w1b, preferred_element_type=_f32) + b1)
            y = jnp.dot(_bf(h), w2b, preferred_element_type=_f32) + b2
            mu = jnp.mean(y, axis=1, keepdims=True)
            var = jnp.mean(jnp.square(y - mu), axis=1, keepdims=True)
            yn = (y - mu) * jax.lax.rsqrt(var + _EPS) * lng + lnb
            gmask = (p0 >= lo) & (p0 < hi)
            out_ref[pl.ds(t * _R, _R), :] = jnp.where(
                gmask, yn, out_ref[pl.ds(t * _R, _R), :])
            return 0

        jax.lax.fori_loop(0, s_ntl[g], tile, 0)

    @pl.when(f == 0)
    def _():
        run(wph1_ref, bph1_ref, wph2_hbm, ph_slots, ph_sems, bph2_ref)

    @pl.when(f == 1)
    def _():
        run(wpr1_ref, bpr1_ref, wpr2_hbm, pr_slots, pr_sems, bpr2_ref)


def _stage_b_body(s_tlo, s_ntl, s_lo, s_hi,
                  oinvc_ref, xin_ref, bt1_ref, bt2_ref,
                  wt1_hbm, wt2_hbm,
                  out_ref,
                  ysort, t1_slots, t2_slots, t1_sems, t2_sems):
    e = pl.program_id(0)

    @pl.when(e == 0)
    def _():
        _copy_expert(wt1_hbm, t1_slots, t1_sems, 0, True)
        _copy_expert(wt2_hbm, t2_slots, t2_sems, 0, True)

    @pl.when(e + 1 < _NE)
    def _():
        _copy_expert(wt1_hbm, t1_slots, t1_sems, e + 1, True)
        _copy_expert(wt2_hbm, t2_slots, t2_sems, e + 1, True)

    _copy_expert(wt1_hbm, t1_slots, t1_sems, e, False)
    _copy_expert(wt2_hbm, t2_slots, t2_sems, e, False)
    slot = jax.lax.rem(e, 2)
    w1b = _bf(t1_slots[slot])
    w2b = _bf(t2_slots[slot])
    b1 = bt1_ref[e]
    b2 = bt2_ref[e]
    lo = s_lo[e]
    hi = s_hi[e]

    def tile(it, _):
        t = s_tlo[e] + it
        xs = xin_ref[pl.ds(t * _R, _R), :]
        h = _gelu(jnp.dot(_bf(xs), w1b, preferred_element_type=_f32) + b1)
        y = jnp.dot(_bf(h), w2b, preferred_element_type=_f32) + b2
        p0 = t * _R + jax.lax.broadcasted_iota(jnp.int32, (_R, 1), 0)
        gmask = (p0 >= lo) & (p0 < hi)
        ysort[pl.ds(t * _R, _R), :] = jnp.where(
            gmask, y, ysort[pl.ds(t * _R, _R), :])
        return 0

    jax.lax.fori_loop(0, s_ntl[e], tile, 0)

    @pl.when(e == _NE - 1)
    def _():
        scat = (oinvc_ref[...] ==
                jax.lax.broadcasted_iota(jnp.int32, (1, _B), 1)
                ).astype(jnp.bfloat16)
        out_ref[...] = jnp.dot(scat, _bf(ysort[...]),
                               preferred_element_type=_f32)


_VMEM_FULL = lambda: pl.BlockSpec(memory_space=pltpu.MemorySpace.VMEM)
_HBM = lambda: pl.BlockSpec(memory_space=pltpu.MemorySpace.HBM)


@jax.jit
def kernel(raw_state, has_proprio, embodiment_idx, W_ph1, b_ph1, W_ph2, b_ph2,
           W_pr1, b_pr1, W_pr2, b_pr2, ln_g, ln_b, W_t1, b_t1, W_t2, b_t2):
    key16 = (embodiment_idx.astype(jnp.int32) * 2
             + has_proprio.astype(jnp.int32))
    oinv, (tlo16, ntl16, lo16, hi16), (tlo8, ntl8, lo8, hi8) = _routing(key16)

    mixed_ln = pl.pallas_call(
        _stage_a_body,
        grid_spec=pltpu.PrefetchScalarGridSpec(
            num_scalar_prefetch=4,
            grid=(_G,),
            in_specs=[_VMEM_FULL() for _ in range(10)] + [_HBM(), _HBM()],
            out_specs=_VMEM_FULL(),
            scratch_shapes=[
                pltpu.VMEM((_B, _S), _f32),
                pltpu.VMEM((2, _H, _D), _f32),
                pltpu.VMEM((2, _H, _D), _f32),
                pltpu.SemaphoreType.DMA((2, _NC)),
                pltpu.SemaphoreType.DMA((2, _NC)),
            ],
        ),
        out_shape=jax.ShapeDtypeStruct((_B, _D), _f32),
        compiler_params=pltpu.CompilerParams(
            dimension_semantics=("arbitrary",),
            vmem_limit_bytes=120 * 1024 * 1024,
        ),
    )(tlo16, ntl16, lo16, hi16,
      oinv.reshape(1, _B), raw_state, W_ph1, W_pr1,
      b_ph1[:, None, :], b_ph2[:, None, :],
      b_pr1[:, None, :], b_pr2[:, None, :],
      ln_g[:, None, :], ln_b[:, None, :],
      W_ph2, W_pr2)

    out = pl.pallas_call(
        _stage_b_body,
        grid_spec=pltpu.PrefetchScalarGridSpec(
            num_scalar_prefetch=4,
            grid=(_NE,),
            in_specs=[_VMEM_FULL() for _ in range(4)] + [_HBM(), _HBM()],
            out_specs=_VMEM_FULL(),
            scratch_shapes=[
                pltpu.VMEM((_B, _D), _f32),
                pltpu.VMEM((2, _D, _H), _f32),
                pltpu.VMEM((2, _H, _D), _f32),
                pltpu.SemaphoreType.DMA((2, _NC)),
                pltpu.SemaphoreType.DMA((2, _NC)),
            ],
        ),
        out_shape=jax.ShapeDtypeStruct((_B, _D), _f32),
        compiler_params=pltpu.CompilerParams(
            dimension_semantics=("arbitrary",),
            vmem_limit_bytes=120 * 1024 * 1024,
        ),
    )(tlo8, ntl8, lo8, hi8,
      oinv.reshape(_B, 1), mixed_ln,
      b_t1[:, None, :], b_t2[:, None, :],
      W_t1, W_t2)

    return out[:, None, :]
